# baseline (speedup 1.0000x reference)
import functools

import jax
import jax.numpy as jnp
from jax import lax
from jax.experimental import pallas as pl
from jax.experimental.pallas import tpu as pltpu
from jax.experimental.pallas import tpu_sc as plsc

B, L, C = 1024, 20, 16
D = 128
N_CHAR = 256
T = B * L

NC, NS, NL = 2, 16, 16
NW = NC * NS
TPW = T // NW
GCHUNK = 128
ACHUNK = 64

_BLK = 512
_NBLK = T // _BLK


def _prep_body(ct_ref, wq_ref, bq_ref, wk_ref, bk_ref, wv_ref, bv_ref,
               m_ref, c0_ref, vt_ref):
    row = lax.broadcasted_iota(jnp.int32, (N_CHAR, D), 0)
    ct0 = jnp.where(row == 0, 0.0, ct_ref[...])
    kt = (jnp.dot(ct0, wk_ref[...], preferred_element_type=jnp.float32)
          + bk_ref[...]) * (D ** -0.5)
    vt_ref[...] = (jnp.dot(ct0, wv_ref[...], preferred_element_type=jnp.float32)
                   + bv_ref[...])
    m_ref[...] = lax.dot_general(wq_ref[...], kt, (((1,), (1,)), ((), ())),
                                 preferred_element_type=jnp.float32)
    c0_ref[...] = lax.dot_general(bq_ref[...], kt, (((1,), (1,)), ((), ())),
                                  preferred_element_type=jnp.float32)


def _prep(char_table, Wq, bq, Wk, bk, Wv, bv):
    return pl.pallas_call(
        _prep_body,
        out_shape=(jax.ShapeDtypeStruct((D, N_CHAR), jnp.float32),
                   jax.ShapeDtypeStruct((1, N_CHAR), jnp.float32),
                   jax.ShapeDtypeStruct((N_CHAR, D), jnp.float32)),
    )(char_table, Wq, bq.reshape(1, D), Wk, bk.reshape(1, D), Wv,
      bv.reshape(1, D))


def _gather_body(table_hbm, idx_hbm, out_hbm, idx_v, rows_v, sem_g, sem_w):
    wid = lax.axis_index("s") * NC + lax.axis_index("c")
    base = wid * TPW
    pltpu.sync_copy(idx_hbm.at[pl.ds(base, TPW)], idx_v)
    gathers = [
        pltpu.async_copy(table_hbm.at[idx_v.at[pl.ds(j * GCHUNK, GCHUNK)]],
                         rows_v.at[pl.ds(j * GCHUNK, GCHUNK)], sem_g)
        for j in range(TPW // GCHUNK)
    ]
    writes = []
    for j, g in enumerate(gathers):
        g.wait()
        writes.append(
            pltpu.async_copy(rows_v.at[pl.ds(j * GCHUNK, GCHUNK)],
                             out_hbm.at[pl.ds(base + j * GCHUNK, GCHUNK)],
                             sem_w))
    for wcp in writes:
        wcp.wait()


def _gather(word_table, word_idx):
    mesh = plsc.VectorSubcoreMesh(core_axis_name="c", subcore_axis_name="s")
    f = functools.partial(
        pl.kernel, mesh=mesh,
        out_type=jax.ShapeDtypeStruct((T, D), jnp.float32),
        scratch_types=[pltpu.VMEM((TPW,), jnp.int32),
                       pltpu.VMEM((TPW, D), jnp.float32),
                       pltpu.SemaphoreType.DMA,
                       pltpu.SemaphoreType.DMA],
        compiler_params=pltpu.CompilerParams(needs_layout_passes=False),
    )(_gather_body)
    return f(word_table, word_idx)


def _score_body(ws_ref, w_ref, m_ref, c0_ref, s_ref):
    wm = jnp.where(ws_ref[...] != 0, w_ref[...], 0.0)
    s = (jnp.dot(wm, m_ref[...], preferred_element_type=jnp.float32)
         + c0_ref[...])
    s_ref[0] = s[:, :D]
    s_ref[1] = s[:, D:]


NP = 4
TP = T // NP
_NBLKP = TP // _BLK


def _score_p(word_idx2d, w, M, c0, h):
    off = h * _NBLKP
    return pl.pallas_call(
        _score_body,
        grid=(_NBLKP,),
        in_specs=[pl.BlockSpec((_BLK, 1), lambda i: (i + off, 0)),
                  pl.BlockSpec((_BLK, D), lambda i: (i + off, 0)),
                  pl.BlockSpec((D, N_CHAR), lambda i: (0, 0)),
                  pl.BlockSpec((1, N_CHAR), lambda i: (0, 0))],
        out_specs=pl.BlockSpec((2, _BLK, D), lambda i: (0, i, 0)),
        out_shape=jax.ShapeDtypeStruct((2, TP, D), jnp.float32),
    )(word_idx2d, w, M, c0)


TPWP = TP // NW
ACH = 32
_NCHP = TPWP // ACH
_HALF = ACH * D
_HALF_SHIFT = _HALF.bit_length() - 1
assert 1 << _HALF_SHIFT == _HALF


def _make_attn_body(h):
    def _attn_body(s_hbm, cid_hbm, w_hbm,
                   s_v0, s_v1, w_v0, w_v1, cid_v,
                   sem_s0, sem_s1, sem_w0, sem_w1):
        wid = lax.axis_index("s") * NC + lax.axis_index("c")
        base = wid * TPWP
        zeros = jnp.zeros((NL,), jnp.float32)
        s_bufs, w_bufs = (s_v0, s_v1), (w_v0, w_v1)
        s_sems, w_sems = (sem_s0, sem_s1), (sem_w0, sem_w1)

        pltpu.sync_copy(cid_hbm.at[pl.ds((h * TP + base) * C, TPWP * C)],
                        cid_v)

        def load_s(j, buf, sem):
            cb = (base + j * ACH) * D
            return (pltpu.async_copy(s_hbm.at[pl.ds(cb, _HALF)],
                                     buf.at[pl.ds(0, _HALF)], sem),
                    pltpu.async_copy(s_hbm.at[pl.ds(TP * D + cb, _HALF)],
                                     buf.at[pl.ds(_HALF, _HALF)], sem))

        s_cps = {0: load_s(0, s_bufs[0], s_sems[0]),
                 1: load_s(1, s_bufs[1], s_sems[1])}
        w_cps = {}

        for j in range(_NCHP):
            b = j & 1
            s_v, w_v = s_bufs[b], w_bufs[b]
            if j >= 2:
                for cp in w_cps[j - 2]:
                    cp.wait()

            def zero_body(z, carry, w_v=w_v):
                w_v[pl.ds(z * NL, NL)] = zeros
                return carry

            lax.fori_loop(0, 2 * _HALF // NL, zero_body, 0, unroll=8)
            for cp in s_cps[j]:
                cp.wait()

            def tok_body(t, carry, s_v=s_v, w_v=w_v, j=j):
                ids = cid_v[pl.ds((j * ACH + t) * C, C)]
                pos = t * D + (ids & (D - 1)) + ((ids >> 7) << _HALF_SHIFT)
                srow = plsc.load_gather(s_v, [pos])
                s = jnp.where(ids == 0, jnp.float32(-1e9), srow)
                m = jnp.max(s)
                e = jnp.exp(s - m)
                plsc.addupdate_scatter(w_v, [pos], e)
                return carry

            lax.fori_loop(0, ACH, tok_body, 0, unroll=4)

            cb = (base + j * ACH) * D
            w_cps[j] = (pltpu.async_copy(w_v.at[pl.ds(0, _HALF)],
                                         w_hbm.at[pl.ds(cb, _HALF)],
                                         w_sems[b]),
                        pltpu.async_copy(w_v.at[pl.ds(_HALF, _HALF)],
                                         w_hbm.at[pl.ds(TP * D + cb, _HALF)],
                                         w_sems[b]))
            if j + 2 < _NCHP:
                s_cps[j + 2] = load_s(j + 2, s_v, s_sems[b])

        for j in (_NCHP - 2, _NCHP - 1):
            for cp in w_cps[j]:
                cp.wait()

    return _attn_body


def _attn_p(S2h, char_ids, h):
    mesh = plsc.VectorSubcoreMesh(core_axis_name="c", subcore_axis_name="s")
    f = functools.partial(
        pl.kernel, mesh=mesh,
        out_type=jax.ShapeDtypeStruct((2 * TP * D,), jnp.float32),
        scratch_types=[pltpu.VMEM((2 * _HALF,), jnp.float32),
                       pltpu.VMEM((2 * _HALF,), jnp.float32),
                       pltpu.VMEM((2 * _HALF,), jnp.float32),
                       pltpu.VMEM((2 * _HALF,), jnp.float32),
                       pltpu.VMEM((TPWP * C,), jnp.int32),
                       pltpu.SemaphoreType.DMA,
                       pltpu.SemaphoreType.DMA,
                       pltpu.SemaphoreType.DMA,
                       pltpu.SemaphoreType.DMA],
        compiler_params=pltpu.CompilerParams(needs_layout_passes=False),
    )(_make_attn_body(h))
    return f(S2h.reshape(2 * TP * D),
             char_ids.reshape(T * C)).reshape(2, TP, D)


_FB = 32
_FT = _FB * L


_NFBP = B // _FB // NP


def _final_body(w_ref, vt_ref, ws_ref, wraw_ref, out_ref):
    w_lo = w_ref[0]
    w_hi = w_ref[1]
    denom = (jnp.sum(w_lo, axis=1, keepdims=True)
             + jnp.sum(w_hi, axis=1, keepdims=True))
    vc = (jnp.dot(w_lo, vt_ref[:D], preferred_element_type=jnp.float32)
          + jnp.dot(w_hi, vt_ref[D:], preferred_element_type=jnp.float32))
    wm = jnp.where(ws_ref[...] != 0, wraw_ref[...], 0.0)
    out_ref[:, :, :D] = wm.reshape(_FB, L, D)
    out_ref[:, :, D:] = (vc / denom).reshape(_FB, L, D)


def _final_chain_body(prev_ref, w_ref, vt_ref, ws_ref, wraw_ref, out_ref):
    del prev_ref
    _final_body(w_ref, vt_ref, ws_ref, wraw_ref, out_ref)


def _final_p(out_prev, W2p, Vt, word_idx2d, w, h):
    boff, toff = h * _NFBP, h * TP // _FT
    w2_spec = pl.BlockSpec((2, _FT, D), lambda i: (0, i, 0))
    vt_spec = pl.BlockSpec((N_CHAR, D), lambda i: (0, 0))
    ws_spec = pl.BlockSpec((_FT, 1), lambda i: (i + toff, 0))
    wr_spec = pl.BlockSpec((_FT, D), lambda i: (i + toff, 0))
    out_spec = pl.BlockSpec((_FB, L, 2 * D), lambda i: (i + boff, 0, 0))
    out_shape = jax.ShapeDtypeStruct((B, L, 2 * D), jnp.float32)
    if h == 0:
        return pl.pallas_call(
            _final_body, grid=(_NFBP,),
            in_specs=[w2_spec, vt_spec, ws_spec, wr_spec],
            out_specs=out_spec, out_shape=out_shape,
        )(W2p, Vt, word_idx2d, w)
    return pl.pallas_call(
        _final_chain_body, grid=(_NFBP,),
        in_specs=[pl.BlockSpec(memory_space=pl.ANY),
                  w2_spec, vt_spec, ws_spec, wr_spec],
        out_specs=out_spec, out_shape=out_shape,
        input_output_aliases={0: 0},
    )(out_prev, W2p, Vt, word_idx2d, w)


def kernel(word_seq, list_char_seq, char_table, word_table, Wq, bq, Wk, bk,
           Wv, bv):
    word_idx = word_seq.reshape(T)
    char_ids = list_char_seq.reshape(T, C)

    M, c0, Vt = _prep(char_table, Wq, bq, Wk, bk, Wv, bv)
    w = _gather(word_table, word_idx)
    idx2d = word_idx.reshape(T, 1)
    S2 = [_score_p(idx2d, w, M, c0, h) for h in range(NP)]
    W2 = [_attn_p(S2[h], char_ids, h) for h in range(NP)]
    out = None
    for h in range(NP):
        out = _final_p(out, W2[h], Vt, idx2d, w, h)
    return out


# speedup vs baseline: 1.0316x; 1.0316x over previous
import functools

import jax
import jax.numpy as jnp
from jax import lax
from jax.experimental import pallas as pl
from jax.experimental.pallas import tpu as pltpu
from jax.experimental.pallas import tpu_sc as plsc

B, L, C = 1024, 20, 16
D = 128
N_CHAR = 256
T = B * L

NC, NS, NL = 2, 16, 16
NW = NC * NS
TPW = T // NW
GCHUNK = 128
ACHUNK = 64

_BLK = 512
_NBLK = T // _BLK


def _prep_body(ct_ref, wq_ref, bq_ref, wk_ref, bk_ref, wv_ref, bv_ref,
               m_ref, c0_ref, vt_ref):
    row = lax.broadcasted_iota(jnp.int32, (N_CHAR, D), 0)
    ct0 = jnp.where(row == 0, 0.0, ct_ref[...])
    kt = (jnp.dot(ct0, wk_ref[...], preferred_element_type=jnp.float32)
          + bk_ref[...]) * (D ** -0.5)
    vt_ref[...] = (jnp.dot(ct0, wv_ref[...], preferred_element_type=jnp.float32)
                   + bv_ref[...])
    m_ref[...] = lax.dot_general(wq_ref[...], kt, (((1,), (1,)), ((), ())),
                                 preferred_element_type=jnp.float32)
    c0_ref[...] = lax.dot_general(bq_ref[...], kt, (((1,), (1,)), ((), ())),
                                  preferred_element_type=jnp.float32)


def _prep(char_table, Wq, bq, Wk, bk, Wv, bv):
    return pl.pallas_call(
        _prep_body,
        out_shape=(jax.ShapeDtypeStruct((D, N_CHAR), jnp.float32),
                   jax.ShapeDtypeStruct((1, N_CHAR), jnp.float32),
                   jax.ShapeDtypeStruct((N_CHAR, D), jnp.float32)),
    )(char_table, Wq, bq.reshape(1, D), Wk, bk.reshape(1, D), Wv,
      bv.reshape(1, D))


_GT2 = T // 2
_GPW = _GT2 // NW
_GCHUNKS = []
_o = 0
while _o < _GPW:
    _GCHUNKS.append((_o, min(GCHUNK, _GPW - _o)))
    _o += GCHUNK


def _make_gather_body(h):
    def _gather_body(table_hbm, idx_hbm, out_hbm, idx_v, rows_v, sem_g, sem_w):
        wid = lax.axis_index("s") * NC + lax.axis_index("c")
        base = wid * _GPW
        pltpu.sync_copy(idx_hbm.at[pl.ds(h * _GT2 + base, _GPW)], idx_v)
        gathers = [
            pltpu.async_copy(table_hbm.at[idx_v.at[pl.ds(o, n)]],
                             rows_v.at[pl.ds(o, n)], sem_g)
            for o, n in _GCHUNKS
        ]
        writes = []
        for (o, n), g in zip(_GCHUNKS, gathers):
            g.wait()
            writes.append(
                pltpu.async_copy(rows_v.at[pl.ds(o, n)],
                                 out_hbm.at[pl.ds(base + o, n)], sem_w))
        for wcp in writes:
            wcp.wait()

    return _gather_body


def _gather_h(word_table, word_idx, h):
    mesh = plsc.VectorSubcoreMesh(core_axis_name="c", subcore_axis_name="s")
    f = functools.partial(
        pl.kernel, mesh=mesh,
        out_type=jax.ShapeDtypeStruct((_GT2, D), jnp.float32),
        scratch_types=[pltpu.VMEM((_GPW,), jnp.int32),
                       pltpu.VMEM((_GPW, D), jnp.float32),
                       pltpu.SemaphoreType.DMA,
                       pltpu.SemaphoreType.DMA],
        compiler_params=pltpu.CompilerParams(needs_layout_passes=False),
    )(_make_gather_body(h))
    return f(word_table, word_idx)


def _score_body(ws_ref, w_ref, m_ref, c0_ref, s_ref):
    wm = jnp.where(ws_ref[...] != 0, w_ref[...], 0.0)
    s = (jnp.dot(wm, m_ref[...], preferred_element_type=jnp.float32)
         + c0_ref[...])
    s_ref[0] = s[:, :D]
    s_ref[1] = s[:, D:]


T2 = T // 2
_NBLK2 = T2 // _BLK


def _score_h(word_idx2d, wh, M, c0, h):
    off = h * _NBLK2
    return pl.pallas_call(
        _score_body,
        grid=(_NBLK2,),
        in_specs=[pl.BlockSpec((_BLK, 1), lambda i: (i + off, 0)),
                  pl.BlockSpec((_BLK, D), lambda i: (i, 0)),
                  pl.BlockSpec((D, N_CHAR), lambda i: (0, 0)),
                  pl.BlockSpec((1, N_CHAR), lambda i: (0, 0))],
        out_specs=pl.BlockSpec((2, _BLK, D), lambda i: (0, i, 0)),
        out_shape=jax.ShapeDtypeStruct((2, T2, D), jnp.float32),
    )(word_idx2d, wh, M, c0)


_NCH = TPW // ACHUNK
_HALF = ACHUNK * D
_HALF_SHIFT = _HALF.bit_length() - 1
assert 1 << _HALF_SHIFT == _HALF


TPW2 = T2 // NW
_NCH2 = TPW2 // ACHUNK


def _make_attn_body(h):
    def _attn_body(s_hbm, cid_hbm, w_hbm,
                   s_v0, s_v1, w_v0, w_v1, cid_v,
                   sem_s0, sem_s1, sem_w0, sem_w1):
        wid = lax.axis_index("s") * NC + lax.axis_index("c")
        base = wid * TPW2
        zeros = jnp.zeros((NL,), jnp.float32)
        s_bufs, w_bufs = (s_v0, s_v1), (w_v0, w_v1)
        s_sems, w_sems = (sem_s0, sem_s1), (sem_w0, sem_w1)

        pltpu.sync_copy(cid_hbm.at[pl.ds((h * T2 + base) * C, TPW2 * C)],
                        cid_v)

        def load_s(j, buf, sem):
            cb = (base + j * ACHUNK) * D
            return (pltpu.async_copy(s_hbm.at[pl.ds(cb, _HALF)],
                                     buf.at[pl.ds(0, _HALF)], sem),
                    pltpu.async_copy(s_hbm.at[pl.ds(T2 * D + cb, _HALF)],
                                     buf.at[pl.ds(_HALF, _HALF)], sem))

        s_cps = {0: load_s(0, s_bufs[0], s_sems[0]),
                 1: load_s(1, s_bufs[1], s_sems[1])}
        w_cps = {}

        for j in range(_NCH2):
            b = j & 1
            s_v, w_v = s_bufs[b], w_bufs[b]
            if j >= 2:
                for cp in w_cps[j - 2]:
                    cp.wait()

            def zero_body(z, carry, w_v=w_v):
                w_v[pl.ds(z * NL, NL)] = zeros
                return carry

            lax.fori_loop(0, 2 * _HALF // NL, zero_body, 0, unroll=8)
            for cp in s_cps[j]:
                cp.wait()

            def tok_body(t, carry, s_v=s_v, w_v=w_v, j=j):
                ids = cid_v[pl.ds((j * ACHUNK + t) * C, C)]
                pos = t * D + (ids & (D - 1)) + ((ids >> 7) << _HALF_SHIFT)
                srow = plsc.load_gather(s_v, [pos])
                s = jnp.where(ids == 0, jnp.float32(-1e9), srow)
                m = jnp.max(s)
                e = jnp.exp(s - m)
                plsc.addupdate_scatter(w_v, [pos], e)
                return carry

            lax.fori_loop(0, ACHUNK, tok_body, 0, unroll=4)

            cb = (base + j * ACHUNK) * D
            w_cps[j] = (pltpu.async_copy(w_v.at[pl.ds(0, _HALF)],
                                         w_hbm.at[pl.ds(cb, _HALF)],
                                         w_sems[b]),
                        pltpu.async_copy(w_v.at[pl.ds(_HALF, _HALF)],
                                         w_hbm.at[pl.ds(T2 * D + cb, _HALF)],
                                         w_sems[b]))
            if j + 2 < _NCH2:
                s_cps[j + 2] = load_s(j + 2, s_v, s_sems[b])

        for j in (_NCH2 - 2, _NCH2 - 1):
            for cp in w_cps[j]:
                cp.wait()

    return _attn_body


def _attn_h(S2h, char_ids, h):
    mesh = plsc.VectorSubcoreMesh(core_axis_name="c", subcore_axis_name="s")
    f = functools.partial(
        pl.kernel, mesh=mesh,
        out_type=jax.ShapeDtypeStruct((2 * T2 * D,), jnp.float32),
        scratch_types=[pltpu.VMEM((2 * _HALF,), jnp.float32),
                       pltpu.VMEM((2 * _HALF,), jnp.float32),
                       pltpu.VMEM((2 * _HALF,), jnp.float32),
                       pltpu.VMEM((2 * _HALF,), jnp.float32),
                       pltpu.VMEM((TPW2 * C,), jnp.int32),
                       pltpu.SemaphoreType.DMA,
                       pltpu.SemaphoreType.DMA,
                       pltpu.SemaphoreType.DMA,
                       pltpu.SemaphoreType.DMA],
        compiler_params=pltpu.CompilerParams(needs_layout_passes=False),
    )(_make_attn_body(h))
    return f(S2h.reshape(2 * T2 * D),
             char_ids.reshape(T * C)).reshape(2, T2, D)


_FB = 32
_FT = _FB * L


_NFB2 = B // _FB // 2


def _final_body(w_ref, vt_ref, ws_ref, wraw_ref, out_ref):
    w_lo = w_ref[0]
    w_hi = w_ref[1]
    denom = (jnp.sum(w_lo, axis=1, keepdims=True)
             + jnp.sum(w_hi, axis=1, keepdims=True))
    vc = (jnp.dot(w_lo, vt_ref[:D], preferred_element_type=jnp.float32)
          + jnp.dot(w_hi, vt_ref[D:], preferred_element_type=jnp.float32))
    wm = jnp.where(ws_ref[...] != 0, wraw_ref[...], 0.0)
    out_ref[:, :, :D] = wm.reshape(_FB, L, D)
    out_ref[:, :, D:] = (vc / denom).reshape(_FB, L, D)


def _final_h0(W2h, Vt, word_idx2d, w):
    return pl.pallas_call(
        _final_body,
        grid=(_NFB2,),
        in_specs=[pl.BlockSpec((2, _FT, D), lambda i: (0, i, 0)),
                  pl.BlockSpec((N_CHAR, D), lambda i: (0, 0)),
                  pl.BlockSpec((_FT, 1), lambda i: (i, 0)),
                  pl.BlockSpec((_FT, D), lambda i: (i, 0))],
        out_specs=pl.BlockSpec((_FB, L, 2 * D), lambda i: (i, 0, 0)),
        out_shape=jax.ShapeDtypeStruct((B, L, 2 * D), jnp.float32),
    )(W2h, Vt, word_idx2d, w)


def _final_h1_body(prev_ref, w_ref, vt_ref, ws_ref, wraw_ref, out_ref):
    del prev_ref
    _final_body(w_ref, vt_ref, ws_ref, wraw_ref, out_ref)


def _final_h1(out0, W2h, Vt, word_idx2d, w):
    return pl.pallas_call(
        _final_h1_body,
        grid=(_NFB2,),
        in_specs=[pl.BlockSpec(memory_space=pl.ANY),
                  pl.BlockSpec((2, _FT, D), lambda i: (0, i, 0)),
                  pl.BlockSpec((N_CHAR, D), lambda i: (0, 0)),
                  pl.BlockSpec((_FT, 1), lambda i: (i + T2 // _FT, 0)),
                  pl.BlockSpec((_FT, D), lambda i: (i, 0))],
        out_specs=pl.BlockSpec((_FB, L, 2 * D), lambda i: (i + _NFB2, 0, 0)),
        out_shape=jax.ShapeDtypeStruct((B, L, 2 * D), jnp.float32),
        input_output_aliases={0: 0},
    )(out0, W2h, Vt, word_idx2d, w)


def kernel(word_seq, list_char_seq, char_table, word_table, Wq, bq, Wk, bk,
           Wv, bv):
    word_idx = word_seq.reshape(T)
    char_ids = list_char_seq.reshape(T, C)

    M, c0, Vt = _prep(char_table, Wq, bq, Wk, bk, Wv, bv)
    w0 = _gather_h(word_table, word_idx, 0)
    w1 = _gather_h(word_table, word_idx, 1)
    idx2d = word_idx.reshape(T, 1)
    S2h0 = _score_h(idx2d, w0, M, c0, 0)
    S2h1 = _score_h(idx2d, w1, M, c0, 1)
    W2h0 = _attn_h(S2h0, char_ids, 0)
    W2h1 = _attn_h(S2h1, char_ids, 1)
    out0 = _final_h0(W2h0, Vt, idx2d, w0)
    return _final_h1(out0, W2h1, Vt, idx2d, w1)
